# baseline (speedup 1.0000x reference)
import jax
import jax.numpy as jnp
from jax import lax
from jax.experimental import pallas as pl
from jax.experimental.pallas import tpu as pltpu
from jax.experimental.pallas import tpu_sc as plsc

V_DIM = 1000000
EMB = 32
BATCH = 16384
CTX = 20

NC = 2
NS = 16
NW = NC * NS

LANES_PER_BLK = 1024
N_BLKS = (V_DIM + LANES_PER_BLK - 1) // LANES_PER_BLK
OUT2_ROWS = N_BLKS * 256
V_PAD = OUT2_ROWS * 4


def _tc_transpose_body(i_ref, o_ref):
    x = i_ref[...]
    eye = jnp.eye(32, dtype=jnp.float32)
    y = lax.dot_general(
        x, eye, (((0,), (0,)), ((), ())), preferred_element_type=jnp.float32
    )
    o_ref[...] = jnp.concatenate(
        [y[0:256], y[256:512], y[512:768], y[768:1024]], axis=1
    )


BPW = BATCH // NW
IDX_PER_W = BPW * CTX
IDX_CHUNK = 128
ROWS_PER_STEP = 32
GATHERS_PER_STEP = ROWS_PER_STEP * CTX // IDX_CHUNK
N_STEPS = BPW // ROWS_PER_STEP
IDX_ROWS_PER_W = IDX_PER_W // IDX_CHUNK


def _tree_sum(vs):
    while len(vs) > 1:
        nxt = [vs[k] + vs[k + 1] for k in range(0, len(vs) - 1, 2)]
        if len(vs) % 2:
            nxt.append(vs[-1])
        vs = nxt
    return vs[0]


def _cbow_body(x_hbm, tab_hbm, out_hbm, idx_v, buf0, buf1, out_v, sem0, sem1):
    wid = lax.axis_index("s") * NC + lax.axis_index("c")

    pltpu.sync_copy(x_hbm.at[pl.ds(wid * IDX_ROWS_PER_W, IDX_ROWS_PER_W)], idx_v)

    bufs = (buf0, buf1)
    sems = (sem0, sem1)

    def permute_row(j):
        for cc in range(IDX_CHUNK // 16):
            v = idx_v[j, pl.ds(16 * cc, 16)]
            w = (v & -1024) | ((v & 255) << 2) | ((v >> 8) & 3)
            idx_v[j, pl.ds(16 * cc, 16)] = w

    def fire(step, slot):
        cps = []
        for j in range(GATHERS_PER_STEP):
            row = step * GATHERS_PER_STEP + j
            permute_row(row)
            cps.append(
                pltpu.async_copy(
                    tab_hbm.at[idx_v.at[row]],
                    bufs[slot].at[pl.ds(j * IDX_CHUNK, IDX_CHUNK)],
                    sems[slot],
                )
            )
        return cps

    def reduce_step(step, slot):
        buf = bufs[slot]
        inv = jnp.float32(1.0 / CTX)

        def row_body(i, carry):
            base = i * CTX
            lo = [buf[base + j, 0:16] for j in range(CTX)]
            hi = [buf[base + j, 16:32] for j in range(CTX)]
            o = step * ROWS_PER_STEP + i
            out_v[o, 0:16] = _tree_sum(lo) * inv
            out_v[o, 16:32] = _tree_sum(hi) * inv
            return carry

        lax.fori_loop(0, ROWS_PER_STEP, row_body, 0)

    inflight = [fire(0, 0), fire(1, 1)]
    for g in range(N_STEPS):
        slot = g % 2
        for cp in inflight[slot]:
            cp.wait()
        reduce_step(g, slot)
        if g + 2 < N_STEPS:
            inflight[slot] = fire(g + 2, slot)

    pltpu.sync_copy(out_v, out_hbm.at[pl.ds(wid * BPW, BPW)])


@jax.jit
def _cbow(x2d, table):
    lin = pl.pallas_call(
        _tc_transpose_body,
        grid=(N_BLKS,),
        in_specs=[pl.BlockSpec((32, LANES_PER_BLK), lambda i: (0, i))],
        out_specs=pl.BlockSpec((256, 128), lambda i: (i, 0)),
        out_shape=jax.ShapeDtypeStruct((OUT2_ROWS, 128), jnp.float32),
    )(table.T)

    tbl = lin.reshape(V_PAD, EMB)

    mesh = plsc.VectorSubcoreMesh(core_axis_name="c", subcore_axis_name="s")
    return pl.kernel(
        _cbow_body,
        out_type=jax.ShapeDtypeStruct((BATCH, EMB), jnp.float32),
        mesh=mesh,
        compiler_params=pltpu.CompilerParams(use_tc_tiling_on_sc=False),
        scratch_types=[
            pltpu.VMEM((IDX_ROWS_PER_W, IDX_CHUNK), jnp.int32),
            pltpu.VMEM((ROWS_PER_STEP * CTX, EMB), jnp.float32),
            pltpu.VMEM((ROWS_PER_STEP * CTX, EMB), jnp.float32),
            pltpu.VMEM((BPW, EMB), jnp.float32),
            pltpu.SemaphoreType.DMA,
            pltpu.SemaphoreType.DMA,
        ],
    )(x2d, tbl)


def kernel(x, table):
    x2d = x.astype(jnp.int32).reshape(BATCH * CTX // IDX_CHUNK, IDX_CHUNK)
    return _cbow(x2d, table)


# speedup vs baseline: 2.3800x; 2.3800x over previous
import jax
import jax.numpy as jnp
from jax import lax
from jax.experimental import pallas as pl
from jax.experimental.pallas import tpu as pltpu
from jax.experimental.pallas import tpu_sc as plsc

V_DIM = 1000000
EMB = 32
BATCH = 16384
CTX = 20

NC = 2
NS = 16
NW = NC * NS

LANES_PER_BLK = 8192
N_BLKS = (V_DIM + LANES_PER_BLK - 1) // LANES_PER_BLK
OUT2_ROWS = N_BLKS * (LANES_PER_BLK // 4)
V_PAD = OUT2_ROWS * 4


def _tc_transpose_body(i_ref, o_ref):
    y = i_ref[...].T
    q = LANES_PER_BLK // 4
    o_ref[...] = jnp.concatenate(
        [y[0 * q : 1 * q], y[1 * q : 2 * q], y[2 * q : 3 * q], y[3 * q : 4 * q]],
        axis=1,
    )


BPW = BATCH // NW
IDX_PER_W = BPW * CTX
IDX_CHUNK = 128
ROWS_PER_STEP = 32
GATHERS_PER_STEP = ROWS_PER_STEP * CTX // IDX_CHUNK
N_STEPS = BPW // ROWS_PER_STEP
IDX_ROWS_PER_W = IDX_PER_W // IDX_CHUNK


def _tree_sum(vs):
    while len(vs) > 1:
        nxt = [vs[k] + vs[k + 1] for k in range(0, len(vs) - 1, 2)]
        if len(vs) % 2:
            nxt.append(vs[-1])
        vs = nxt
    return vs[0]


def _cbow_body(x_hbm, tab_hbm, out_hbm, idx_v, buf0, buf1, out_v, sem0, sem1):
    wid = lax.axis_index("s") * NC + lax.axis_index("c")

    pltpu.sync_copy(x_hbm.at[pl.ds(wid * IDX_ROWS_PER_W, IDX_ROWS_PER_W)], idx_v)

    bufs = (buf0, buf1)
    sems = (sem0, sem1)

    def permute_row(j):
        for cc in range(IDX_CHUNK // 16):
            v = idx_v[j, pl.ds(16 * cc, 16)]
            w = (v & -LANES_PER_BLK) | ((v & (LANES_PER_BLK // 4 - 1)) << 2) | ((v >> 11) & 3)
            idx_v[j, pl.ds(16 * cc, 16)] = w

    def fire(step, slot):
        cps = []
        for j in range(GATHERS_PER_STEP):
            row = step * GATHERS_PER_STEP + j
            permute_row(row)
            cps.append(
                pltpu.async_copy(
                    tab_hbm.at[idx_v.at[row]],
                    bufs[slot].at[pl.ds(j * IDX_CHUNK, IDX_CHUNK)],
                    sems[slot],
                )
            )
        return cps

    def reduce_step(step, slot):
        buf = bufs[slot]
        inv = jnp.float32(1.0 / CTX)

        def row_body(i, carry):
            base = i * CTX
            lo = [buf[base + j, 0:16] for j in range(CTX)]
            hi = [buf[base + j, 16:32] for j in range(CTX)]
            o = step * ROWS_PER_STEP + i
            out_v[o, 0:16] = _tree_sum(lo) * inv
            out_v[o, 16:32] = _tree_sum(hi) * inv
            return carry

        lax.fori_loop(0, ROWS_PER_STEP, row_body, 0)

    inflight = [fire(0, 0), fire(1, 1)]
    for g in range(N_STEPS):
        slot = g % 2
        for cp in inflight[slot]:
            cp.wait()
        reduce_step(g, slot)
        if g + 2 < N_STEPS:
            inflight[slot] = fire(g + 2, slot)

    pltpu.sync_copy(out_v, out_hbm.at[pl.ds(wid * BPW, BPW)])


@jax.jit
def _cbow(x2d, table):
    lin = pl.pallas_call(
        _tc_transpose_body,
        grid=(N_BLKS,),
        in_specs=[pl.BlockSpec((32, LANES_PER_BLK), lambda i: (0, i))],
        out_specs=pl.BlockSpec((LANES_PER_BLK // 4, 128), lambda i: (i, 0)),
        out_shape=jax.ShapeDtypeStruct((OUT2_ROWS, 128), jnp.float32),
    )(table.T)

    tbl = lin.reshape(V_PAD, EMB)

    mesh = plsc.VectorSubcoreMesh(core_axis_name="c", subcore_axis_name="s")
    return pl.kernel(
        _cbow_body,
        out_type=jax.ShapeDtypeStruct((BATCH, EMB), jnp.float32),
        mesh=mesh,
        compiler_params=pltpu.CompilerParams(use_tc_tiling_on_sc=False),
        scratch_types=[
            pltpu.VMEM((IDX_ROWS_PER_W, IDX_CHUNK), jnp.int32),
            pltpu.VMEM((ROWS_PER_STEP * CTX, EMB), jnp.float32),
            pltpu.VMEM((ROWS_PER_STEP * CTX, EMB), jnp.float32),
            pltpu.VMEM((BPW, EMB), jnp.float32),
            pltpu.SemaphoreType.DMA,
            pltpu.SemaphoreType.DMA,
        ],
    )(x2d, tbl)


def kernel(x, table):
    x2d = x.astype(jnp.int32).reshape(BATCH * CTX // IDX_CHUNK, IDX_CHUNK)
    return _cbow(x2d, table)


# speedup vs baseline: 2.4267x; 1.0196x over previous
import jax
import jax.numpy as jnp
from jax import lax
from jax.experimental import pallas as pl
from jax.experimental.pallas import tpu as pltpu
from jax.experimental.pallas import tpu_sc as plsc

V_DIM = 1000000
EMB = 32
BATCH = 16384
CTX = 20

NC = 2
NS = 16
NW = NC * NS

LANES_PER_BLK = 32768
N_BLKS = (V_DIM + LANES_PER_BLK - 1) // LANES_PER_BLK
OUT2_ROWS = N_BLKS * (LANES_PER_BLK // 4)
V_PAD = OUT2_ROWS * 4


def _tc_transpose_body(i_ref, o_ref):
    y = i_ref[...].T
    q = LANES_PER_BLK // 4
    o_ref[...] = jnp.concatenate(
        [y[0 * q : 1 * q], y[1 * q : 2 * q], y[2 * q : 3 * q], y[3 * q : 4 * q]],
        axis=1,
    )


BPW = BATCH // NW
IDX_PER_W = BPW * CTX
IDX_CHUNK = 128
ROWS_PER_STEP = 32
GATHERS_PER_STEP = ROWS_PER_STEP * CTX // IDX_CHUNK
N_STEPS = BPW // ROWS_PER_STEP
IDX_ROWS_PER_W = IDX_PER_W // IDX_CHUNK


def _tree_sum(vs):
    while len(vs) > 1:
        nxt = [vs[k] + vs[k + 1] for k in range(0, len(vs) - 1, 2)]
        if len(vs) % 2:
            nxt.append(vs[-1])
        vs = nxt
    return vs[0]


def _cbow_body(x_hbm, tab_hbm, out_hbm, idx_v, buf0, buf1, out_v, sem0, sem1):
    wid = lax.axis_index("s") * NC + lax.axis_index("c")

    pltpu.sync_copy(x_hbm.at[pl.ds(wid * IDX_ROWS_PER_W, IDX_ROWS_PER_W)], idx_v)

    bufs = (buf0, buf1)
    sems = (sem0, sem1)

    def permute_row(j):
        for cc in range(IDX_CHUNK // 16):
            v = idx_v[j, pl.ds(16 * cc, 16)]
            w = (v & -LANES_PER_BLK) | ((v & (LANES_PER_BLK // 4 - 1)) << 2) | ((v >> 13) & 3)
            idx_v[j, pl.ds(16 * cc, 16)] = w

    def fire(step, slot):
        cps = []
        for j in range(GATHERS_PER_STEP):
            row = step * GATHERS_PER_STEP + j
            permute_row(row)
            cps.append(
                pltpu.async_copy(
                    tab_hbm.at[idx_v.at[row]],
                    bufs[slot].at[pl.ds(j * IDX_CHUNK, IDX_CHUNK)],
                    sems[slot],
                )
            )
        return cps

    def reduce_step(step, slot):
        buf = bufs[slot]
        inv = jnp.float32(1.0 / CTX)

        def row_body(i, carry):
            base = i * CTX
            lo = [buf[base + j, 0:16] for j in range(CTX)]
            hi = [buf[base + j, 16:32] for j in range(CTX)]
            o = step * ROWS_PER_STEP + i
            out_v[o, 0:16] = _tree_sum(lo) * inv
            out_v[o, 16:32] = _tree_sum(hi) * inv
            return carry

        lax.fori_loop(0, ROWS_PER_STEP, row_body, 0)

    inflight = [fire(0, 0), fire(1, 1)]
    for g in range(N_STEPS):
        slot = g % 2
        for cp in inflight[slot]:
            cp.wait()
        reduce_step(g, slot)
        if g + 2 < N_STEPS:
            inflight[slot] = fire(g + 2, slot)

    pltpu.sync_copy(out_v, out_hbm.at[pl.ds(wid * BPW, BPW)])


@jax.jit
def _cbow(x2d, table):
    lin = pl.pallas_call(
        _tc_transpose_body,
        grid=(N_BLKS,),
        in_specs=[pl.BlockSpec((32, LANES_PER_BLK), lambda i: (0, i))],
        out_specs=pl.BlockSpec((LANES_PER_BLK // 4, 128), lambda i: (i, 0)),
        out_shape=jax.ShapeDtypeStruct((OUT2_ROWS, 128), jnp.float32),
    )(table.T)

    tbl = lin.reshape(V_PAD, EMB)

    mesh = plsc.VectorSubcoreMesh(core_axis_name="c", subcore_axis_name="s")
    return pl.kernel(
        _cbow_body,
        out_type=jax.ShapeDtypeStruct((BATCH, EMB), jnp.float32),
        mesh=mesh,
        compiler_params=pltpu.CompilerParams(use_tc_tiling_on_sc=False),
        scratch_types=[
            pltpu.VMEM((IDX_ROWS_PER_W, IDX_CHUNK), jnp.int32),
            pltpu.VMEM((ROWS_PER_STEP * CTX, EMB), jnp.float32),
            pltpu.VMEM((ROWS_PER_STEP * CTX, EMB), jnp.float32),
            pltpu.VMEM((BPW, EMB), jnp.float32),
            pltpu.SemaphoreType.DMA,
            pltpu.SemaphoreType.DMA,
        ],
    )(x2d, tbl)


def kernel(x, table):
    x2d = x.astype(jnp.int32).reshape(BATCH * CTX // IDX_CHUNK, IDX_CHUNK)
    return _cbow(x2d, table)
